# baseline (speedup 1.0000x reference)
import jax
import jax.numpy as jnp
from jax import lax
from jax.experimental import pallas as pl
from jax.experimental.pallas import tpu as pltpu
from jax.experimental.pallas import tpu_sc as plsc

_N = 10000
_D = 128
_B = 4096
_T = 12
_S1 = 5
_NW = 32
_CH = 64

_NSPLIT = 2
_BC = _B // _NSPLIT

_NA = (_T * _BC) // _NW // _CH
_NH = (_T * _S1 * _BC) // _NW // _CH
_ROWS_A = _NA * _CH
_ROWS_H = _NH * _CH


def _sc_body(xf, idxa, idx1, idx2a, idx2b, a0o, hmo,
             iva, iv1, iv2a, iv2b, bufs, gsem, osem):
    cid = lax.axis_index("c")
    sid = lax.axis_index("s")
    wid = sid * 2 + cid

    pltpu.sync_copy(idxa.at[wid], iva)
    pltpu.sync_copy(idx1.at[wid], iv1)
    pltpu.sync_copy(idx2a.at[wid], iv2a)
    pltpu.sync_copy(idx2b.at[wid], iv2b)

    def plain_phase(iv, n_groups, dst_slice):
        def fire(g, par):
            for i in range(2):
                pltpu.make_async_copy(
                    xf.at[iv.at[g * 2 + i]], bufs.at[par * 2 + i], gsem).start()

        def waitg(par):
            for i in range(2):
                pltpu.make_async_copy(
                    xf.at[iv.at[0]], bufs.at[par * 2 + i], gsem).wait()

        def firew(g, par):
            for i in range(2):
                pltpu.make_async_copy(
                    bufs.at[par * 2 + i], dst_slice(g * 2 + i), osem).start()

        def waitw(par):
            for i in range(2):
                pltpu.make_async_copy(
                    bufs.at[par * 2 + i], dst_slice(0), osem).wait()

        fire(0, 0)
        for g in range(n_groups):
            par = g % 2
            waitg(par)
            if g + 1 < n_groups:
                if g >= 1:
                    waitw(1 - par)
                fire(g + 1, 1 - par)
            firew(g, par)
        if n_groups >= 2:
            waitw((n_groups - 2) % 2)
        waitw((n_groups - 1) % 2)

    plain_phase(iva, _NA // 2,
                lambda ch: a0o.at[pl.ds(pl.multiple_of(wid * _ROWS_A + ch * _CH, _CH), _CH)])
    plain_phase(iv1, _NH // 2,
                lambda ch: hmo.at[pl.ds(pl.multiple_of(wid * _ROWS_H + ch * _CH, _CH), _CH),
                                  pl.ds(0, _D)])

    def m_fire(g, par):
        pltpu.make_async_copy(xf.at[iv2a.at[g]], bufs.at[par * 2], gsem).start()
        pltpu.make_async_copy(xf.at[iv2b.at[g]], bufs.at[par * 2 + 1], gsem).start()

    def m_consume(g, par):
        for i in range(2):
            pltpu.make_async_copy(
                xf.at[iv2a.at[0]], bufs.at[par * 2 + i], gsem).wait()

        def addrow(k, c2):
            for cg in range(8):
                v = bufs[par * 2 + 1, k, pl.ds(cg * 16, 16)]
                plsc.addupdate(bufs.at[par * 2, k, pl.ds(cg * 16, 16)], v)
            return c2
        lax.fori_loop(0, _CH, addrow, 0)
        pltpu.make_async_copy(
            bufs.at[par * 2],
            hmo.at[pl.ds(pl.multiple_of(wid * _ROWS_H + g * _CH, _CH), _CH),
                   pl.ds(_D, _D)],
            osem).start()
        pltpu.make_async_copy(
            bufs.at[par * 2],
            hmo.at[pl.ds(pl.multiple_of(wid * _ROWS_H, _CH), _CH),
                   pl.ds(_D, _D)], osem).wait()

    m_fire(0, 0)

    def m_body(kk, c):
        g0 = 2 * kk
        m_fire(g0 + 1, 1); m_consume(g0, 0)
        m_fire(g0 + 2, 0); m_consume(g0 + 1, 1)
        return c
    lax.fori_loop(0, _NH // 2 - 1, m_body, 0)
    m_fire(_NH - 1, 1); m_consume(_NH - 2, 0)
    m_consume(_NH - 1, 1)


def _sc_gather(xf, idxa2, idx12, idx2a2, idx2b2):
    f32 = jnp.float32
    run = pl.kernel(
        _sc_body,
        out_type=[
            jax.ShapeDtypeStruct((_T * _BC, _D), f32),
            jax.ShapeDtypeStruct((_T * _S1 * _BC, 2 * _D), f32),
        ],
        mesh=plsc.VectorSubcoreMesh(
            core_axis_name="c", subcore_axis_name="s", num_cores=2, num_subcores=16),
        scratch_types=[
            pltpu.VMEM((_NA, _CH), jnp.int32),
            pltpu.VMEM((_NH, _CH), jnp.int32),
            pltpu.VMEM((_NH, _CH), jnp.int32),
            pltpu.VMEM((_NH, _CH), jnp.int32),
            pltpu.VMEM((4, _CH, _D), f32),
            pltpu.SemaphoreType.DMA,
            pltpu.SemaphoreType.DMA,
        ],
    )
    return run(xf, idxa2, idx12, idx2a2, idx2b2)


_BR = 1024
_NBLK = _BC // _BR


def _spike(v):
    return (v >= 1.0).astype(jnp.float32)


def _tc_body(a0_ref, hm_ref, ws1_ref, wn1_ref, b1_ref, ws2_ref, wn2_ref,
             b2_ref, pw_ref, pb1_ref, pb2_ref, pb3_ref, wm_ref, bm_ref,
             out_ref, acc_ref):
    t = pl.program_id(1)

    @pl.when(t == 0)
    def _():
        acc_ref[...] = jnp.zeros_like(acc_ref)

    a0 = a0_ref[0]
    hm = hm_ref[0]
    h1sum = (hm[0, :, :_D] + hm[1, :, :_D] + hm[2, :, :_D]
             + hm[3, :, :_D] + hm[4, :, :_D])
    m1 = h1sum / 5.0
    cat1 = jnp.concatenate([a0, m1], axis=1)

    def mm(a, w):
        return jnp.dot(a, w, preferred_element_type=jnp.float32)

    def chan(c):
        ws1 = ws1_ref[c]
        wn1 = wn1_ref[c]
        b1 = b1_ref[c]
        w1g0 = jnp.concatenate([ws1, wn1], axis=0)
        w1g1 = jnp.concatenate([ws1, wn1 * 0.5], axis=0)
        g0 = _spike(mm(cat1, w1g0) + b1)
        p = jnp.zeros((_BR, _D), jnp.float32)
        for s in range(_S1):
            p = p + _spike(mm(hm[s], w1g1) + b1)
        cat3 = jnp.concatenate([g0, p / 5.0], axis=1)
        w2 = jnp.concatenate([ws2_ref[c], wn2_ref[c]], axis=0)
        e = _spike(mm(cat3, w2) + b2_ref[c])
        acc_ref[...] += e * pw_ref[0, c]

    chan(0)
    pl.when(t % 2 == 0)(lambda: chan(1))
    pl.when(t % 3 == 0)(lambda: chan(2))

    @pl.when(t == _T - 1)
    def _():
        pbs = pb1_ref[...] + pb2_ref[...] + pb3_ref[...]
        emb = (acc_ref[...] + pbs) / 3.0
        out_ref[...] = jnp.dot(emb, wm_ref[...],
                               preferred_element_type=jnp.float32) + bm_ref[...]


def _tc_forward(a0, hm, Ws1, Wn1, b1r, Ws2, Wn2, b2r, pw, pb1r, pb2r, pb3r,
                Wm, bmr, interpret=False):
    grid = (_NBLK, _T)
    full = lambda shape: pl.BlockSpec(shape, lambda i, t: (0,) * len(shape))
    return pl.pallas_call(
        _tc_body,
        grid=grid,
        in_specs=[
            pl.BlockSpec((1, _BR, _D), lambda i, t: (t, i, 0)),
            pl.BlockSpec((1, _S1, _BR, 2 * _D), lambda i, t: (t, 0, i, 0)),
            full((3, _D, _D)),
            full((3, _D, _D)),
            full((3, 1, _D)),
            full((3, _D, 64)),
            full((3, _D, 64)),
            full((3, 1, 64)),
            pl.BlockSpec((1, 3, 1, 64), lambda i, t: (t, 0, 0, 0)),
            full((1, 64)),
            full((1, 64)),
            full((1, 64)),
            full((64, 64)),
            full((1, 64)),
        ],
        out_specs=pl.BlockSpec((_BR, 64), lambda i, t: (i, 0)),
        out_shape=jax.ShapeDtypeStruct((_BC, 64), jnp.float32),
        scratch_shapes=[pltpu.VMEM((_BR, 64), jnp.float32)],
        interpret=interpret,
    )(a0, hm, Ws1, Wn1, b1r, Ws2, Wn2, b2r, pw, pb1r, pb2r, pb3r, Wm, bmr)


def kernel(x, Ws1, Wn1, b1, Ws2, Wn2, b2, pw1, pb1, pw2, pb2, pw3, pb3, Wm, bm,
           nodes, nbr1, nbr2):
    i32 = jnp.int32
    offs = (jnp.arange(_T, dtype=i32) * _N)

    idxa = nodes.astype(i32)[None, :] + offs[:, None]
    idx1 = nbr1.astype(i32).transpose(0, 2, 1) + offs[:, None, None]
    n2r = nbr2.astype(i32).reshape(_T, _B, _S1, 2)
    idx2a = n2r[..., 0].transpose(0, 2, 1) + offs[:, None, None]
    idx2b = n2r[..., 1].transpose(0, 2, 1) + offs[:, None, None]

    xf = x.reshape(_T * _N, _D)

    pw = jnp.zeros((_T, 3, 64), jnp.float32)
    pw = pw.at[:, 0].set(pw1.T)
    pw = pw.at[0::2, 1].set(pw2.T)
    pw = pw.at[0::3, 2].set(pw3.T)
    pw = pw.reshape(_T, 3, 1, 64)

    outs = []
    for h in range(_NSPLIT):
        bs = slice(h * _BC, (h + 1) * _BC)
        a0f, hmf = _sc_gather(
            xf,
            idxa[:, bs].reshape(_NW, _NA, _CH),
            idx1[:, :, bs].reshape(_NW, _NH, _CH),
            idx2a[:, :, bs].reshape(_NW, _NH, _CH),
            idx2b[:, :, bs].reshape(_NW, _NH, _CH),
        )
        a0 = a0f.reshape(_T, _BC, _D)
        hm = hmf.reshape(_T, _S1, _BC, 2 * _D)
        outs.append(_tc_forward(
            a0, hm, Ws1, Wn1, b1.reshape(3, 1, _D), Ws2, Wn2,
            b2.reshape(3, 1, 64), pw, pb1.reshape(1, 64), pb2.reshape(1, 64),
            pb3.reshape(1, 64), Wm, bm.reshape(1, 64)))
    return jnp.concatenate(outs, axis=0)


# speedup vs baseline: 1.0837x; 1.0837x over previous
import jax
import jax.numpy as jnp
from jax import lax
from jax.experimental import pallas as pl
from jax.experimental.pallas import tpu as pltpu
from jax.experimental.pallas import tpu_sc as plsc

_N = 10000
_D = 128
_B = 4096
_T = 12
_S1 = 5
_NW = 32
_CH = 128

_NSPLIT = 4
_BC = _B // _NSPLIT

_NA = (_T * _BC) // _NW // _CH
_NH = (_T * _S1 * _BC) // _NW // _CH
_ROWS_A = _NA * _CH
_ROWS_H = _NH * _CH


def _sc_body(xf, idxa, idx1, idx2a, idx2b, a0o, hmo,
             iva, iv1, iv2a, iv2b, bufs, gsem, osem):
    cid = lax.axis_index("c")
    sid = lax.axis_index("s")
    wid = sid * 2 + cid

    pltpu.sync_copy(idxa.at[wid], iva)
    pltpu.sync_copy(idx1.at[wid], iv1)
    pltpu.sync_copy(idx2a.at[wid], iv2a)
    pltpu.sync_copy(idx2b.at[wid], iv2b)

    def plain_phase(iv, n_chunks, dst_slice):
        groups = [(2 * i, 2) for i in range(n_chunks // 2)]
        if n_chunks % 2:
            groups.append((n_chunks - 1, 1))
        n = len(groups)

        def fire(gi, par):
            g0, cnt = groups[gi]
            for i in range(cnt):
                pltpu.make_async_copy(
                    xf.at[iv.at[g0 + i]], bufs.at[par * 2 + i], gsem).start()

        def waitg(gi, par):
            for i in range(groups[gi][1]):
                pltpu.make_async_copy(
                    xf.at[iv.at[0]], bufs.at[par * 2 + i], gsem).wait()

        def firew(gi, par):
            g0, cnt = groups[gi]
            for i in range(cnt):
                pltpu.make_async_copy(
                    bufs.at[par * 2 + i], dst_slice(g0 + i), osem).start()

        def waitw(gi, par):
            for i in range(groups[gi][1]):
                pltpu.make_async_copy(
                    bufs.at[par * 2 + i], dst_slice(0), osem).wait()

        fire(0, 0)
        for gi in range(n):
            par = gi % 2
            waitg(gi, par)
            if gi + 1 < n:
                if gi >= 1:
                    waitw(gi - 1, 1 - par)
                fire(gi + 1, 1 - par)
            firew(gi, par)
        if n >= 2:
            waitw(n - 2, n % 2)
        waitw(n - 1, (n - 1) % 2)

    plain_phase(iva, _NA,
                lambda ch: a0o.at[pl.ds(pl.multiple_of(wid * _ROWS_A + ch * _CH, _CH), _CH)])
    plain_phase(iv1, _NH,
                lambda ch: hmo.at[pl.ds(pl.multiple_of(wid * _ROWS_H + ch * _CH, _CH), _CH),
                                  pl.ds(0, _D)])

    def m_fire(g, par):
        pltpu.make_async_copy(xf.at[iv2a.at[g]], bufs.at[par * 2], gsem).start()
        pltpu.make_async_copy(xf.at[iv2b.at[g]], bufs.at[par * 2 + 1], gsem).start()

    def m_consume(g, par):
        for i in range(2):
            pltpu.make_async_copy(
                xf.at[iv2a.at[0]], bufs.at[par * 2 + i], gsem).wait()

        def addrow(k, c2):
            for cg in range(8):
                v = bufs[par * 2 + 1, k, pl.ds(cg * 16, 16)]
                plsc.addupdate(bufs.at[par * 2, k, pl.ds(cg * 16, 16)], v)
            return c2
        lax.fori_loop(0, _CH, addrow, 0)
        pltpu.make_async_copy(
            bufs.at[par * 2],
            hmo.at[pl.ds(pl.multiple_of(wid * _ROWS_H + g * _CH, _CH), _CH),
                   pl.ds(_D, _D)],
            osem).start()
        pltpu.make_async_copy(
            bufs.at[par * 2],
            hmo.at[pl.ds(pl.multiple_of(wid * _ROWS_H, _CH), _CH),
                   pl.ds(_D, _D)], osem).wait()

    m_fire(0, 0)

    def m_body(kk, c):
        g0 = 2 * kk
        m_fire(g0 + 1, 1); m_consume(g0, 0)
        m_fire(g0 + 2, 0); m_consume(g0 + 1, 1)
        return c
    if _NH % 2 == 0:
        lax.fori_loop(0, _NH // 2 - 1, m_body, 0)
        m_fire(_NH - 1, 1); m_consume(_NH - 2, 0)
        m_consume(_NH - 1, 1)
    else:
        lax.fori_loop(0, (_NH - 1) // 2 - 1, m_body, 0)
        m_fire(_NH - 2, 1); m_consume(_NH - 3, 0)
        m_fire(_NH - 1, 0); m_consume(_NH - 2, 1)
        m_consume(_NH - 1, 0)


def _sc_gather(xf, idxa2, idx12, idx2a2, idx2b2):
    f32 = jnp.float32
    run = pl.kernel(
        _sc_body,
        out_type=[
            jax.ShapeDtypeStruct((_T * _BC, _D), f32),
            jax.ShapeDtypeStruct((_T * _S1 * _BC, 2 * _D), f32),
        ],
        mesh=plsc.VectorSubcoreMesh(
            core_axis_name="c", subcore_axis_name="s", num_cores=2, num_subcores=16),
        scratch_types=[
            pltpu.VMEM((_NA, _CH), jnp.int32),
            pltpu.VMEM((_NH, _CH), jnp.int32),
            pltpu.VMEM((_NH, _CH), jnp.int32),
            pltpu.VMEM((_NH, _CH), jnp.int32),
            pltpu.VMEM((4, _CH, _D), f32),
            pltpu.SemaphoreType.DMA,
            pltpu.SemaphoreType.DMA,
        ],
    )
    return run(xf, idxa2, idx12, idx2a2, idx2b2)


_BR = 1024
_NBLK = _BC // _BR


def _spike(v):
    return (v >= 1.0).astype(jnp.float32)


def _tc_body(a0_ref, hm_ref, ws1_ref, wn1_ref, b1_ref, ws2_ref, wn2_ref,
             b2_ref, pw_ref, pb1_ref, pb2_ref, pb3_ref, wm_ref, bm_ref,
             out_ref, acc_ref):
    t = pl.program_id(1)

    @pl.when(t == 0)
    def _():
        acc_ref[...] = jnp.zeros_like(acc_ref)

    a0 = a0_ref[0]
    hm = hm_ref[0]
    h1sum = (hm[0, :, :_D] + hm[1, :, :_D] + hm[2, :, :_D]
             + hm[3, :, :_D] + hm[4, :, :_D])
    m1 = h1sum / 5.0
    cat1 = jnp.concatenate([a0, m1], axis=1)

    def mm(a, w):
        return jnp.dot(a, w, preferred_element_type=jnp.float32)

    def chan(c):
        ws1 = ws1_ref[c]
        wn1 = wn1_ref[c]
        b1 = b1_ref[c]
        w1g0 = jnp.concatenate([ws1, wn1], axis=0)
        w1g1 = jnp.concatenate([ws1, wn1 * 0.5], axis=0)
        g0 = _spike(mm(cat1, w1g0) + b1)
        p = jnp.zeros((_BR, _D), jnp.float32)
        for s in range(_S1):
            p = p + _spike(mm(hm[s], w1g1) + b1)
        cat3 = jnp.concatenate([g0, p / 5.0], axis=1)
        w2 = jnp.concatenate([ws2_ref[c], wn2_ref[c]], axis=0)
        e = _spike(mm(cat3, w2) + b2_ref[c])
        acc_ref[...] += e * pw_ref[0, c]

    chan(0)
    pl.when(t % 2 == 0)(lambda: chan(1))
    pl.when(t % 3 == 0)(lambda: chan(2))

    @pl.when(t == _T - 1)
    def _():
        pbs = pb1_ref[...] + pb2_ref[...] + pb3_ref[...]
        emb = (acc_ref[...] + pbs) / 3.0
        out_ref[...] = jnp.dot(emb, wm_ref[...],
                               preferred_element_type=jnp.float32) + bm_ref[...]


def _tc_forward(a0, hm, Ws1, Wn1, b1r, Ws2, Wn2, b2r, pw, pb1r, pb2r, pb3r,
                Wm, bmr, interpret=False):
    grid = (_NBLK, _T)
    full = lambda shape: pl.BlockSpec(shape, lambda i, t: (0,) * len(shape))
    return pl.pallas_call(
        _tc_body,
        grid=grid,
        in_specs=[
            pl.BlockSpec((1, _BR, _D), lambda i, t: (t, i, 0)),
            pl.BlockSpec((1, _S1, _BR, 2 * _D), lambda i, t: (t, 0, i, 0)),
            full((3, _D, _D)),
            full((3, _D, _D)),
            full((3, 1, _D)),
            full((3, _D, 64)),
            full((3, _D, 64)),
            full((3, 1, 64)),
            pl.BlockSpec((1, 3, 1, 64), lambda i, t: (t, 0, 0, 0)),
            full((1, 64)),
            full((1, 64)),
            full((1, 64)),
            full((64, 64)),
            full((1, 64)),
        ],
        out_specs=pl.BlockSpec((_BR, 64), lambda i, t: (i, 0)),
        out_shape=jax.ShapeDtypeStruct((_BC, 64), jnp.float32),
        scratch_shapes=[pltpu.VMEM((_BR, 64), jnp.float32)],
        interpret=interpret,
    )(a0, hm, Ws1, Wn1, b1r, Ws2, Wn2, b2r, pw, pb1r, pb2r, pb3r, Wm, bmr)


def kernel(x, Ws1, Wn1, b1, Ws2, Wn2, b2, pw1, pb1, pw2, pb2, pw3, pb3, Wm, bm,
           nodes, nbr1, nbr2):
    i32 = jnp.int32
    offs = (jnp.arange(_T, dtype=i32) * _N)

    idxa = nodes.astype(i32)[None, :] + offs[:, None]
    idx1 = nbr1.astype(i32).transpose(0, 2, 1) + offs[:, None, None]
    n2r = nbr2.astype(i32).reshape(_T, _B, _S1, 2)
    idx2a = n2r[..., 0].transpose(0, 2, 1) + offs[:, None, None]
    idx2b = n2r[..., 1].transpose(0, 2, 1) + offs[:, None, None]

    xf = x.reshape(_T * _N, _D)

    pw = jnp.zeros((_T, 3, 64), jnp.float32)
    pw = pw.at[:, 0].set(pw1.T)
    pw = pw.at[0::2, 1].set(pw2.T)
    pw = pw.at[0::3, 2].set(pw3.T)
    pw = pw.reshape(_T, 3, 1, 64)

    outs = []
    for h in range(_NSPLIT):
        bs = slice(h * _BC, (h + 1) * _BC)
        a0f, hmf = _sc_gather(
            xf,
            idxa[:, bs].reshape(_NW, _NA, _CH),
            idx1[:, :, bs].reshape(_NW, _NH, _CH),
            idx2a[:, :, bs].reshape(_NW, _NH, _CH),
            idx2b[:, :, bs].reshape(_NW, _NH, _CH),
        )
        a0 = a0f.reshape(_T, _BC, _D)
        hm = hmf.reshape(_T, _S1, _BC, 2 * _D)
        outs.append(_tc_forward(
            a0, hm, Ws1, Wn1, b1.reshape(3, 1, _D), Ws2, Wn2,
            b2.reshape(3, 1, 64), pw, pb1.reshape(1, 64), pb2.reshape(1, 64),
            pb3.reshape(1, 64), Wm, bm.reshape(1, 64)))
    return jnp.concatenate(outs, axis=0)
